# baseline (speedup 1.0000x reference)
import functools

import jax
import jax.numpy as jnp
from jax import lax
from jax.experimental import pallas as pl
from jax.experimental.pallas import tpu as pltpu
from jax.experimental.pallas import tpu_sc as plsc

N = 50000
E = 800000
D = 64
DH = 32
HOPS = 3

NUM_CORES = 2
NUM_TILES = 16
CH = 128
KR = 16
SUPER = 26
NB = 4
CROWS_PER_TILE = KR * SUPER
CROWS = NUM_TILES * CROWS_PER_TILE
MAX_CROW = CROWS - KR
E_PAD = CROWS * CH
N_PAD = 50176
ROWS_PER_TILE = N_PAD // NUM_TILES
N_PAD4 = N_PAD // 4
ZROWS = 56

_mesh = plsc.VectorSubcoreMesh(core_axis_name="c", subcore_axis_name="s")


@functools.partial(
    pl.kernel,
    out_type=jax.ShapeDtypeStruct((NUM_CORES, N_PAD, DH), jnp.float32),
    mesh=_mesh,
    compiler_params=pltpu.CompilerParams(use_tc_tiling_on_sc=False),
    scratch_types=[
        pltpu.VMEM_SHARED((N_PAD, DH), jnp.float32),
        [pltpu.VMEM((KR, CH), jnp.int32) for _ in range(2)],
        [pltpu.VMEM((KR, CH), jnp.int32) for _ in range(2)],
        [pltpu.VMEM((KR * CH,), jnp.float32) for _ in range(2)],
        [pltpu.VMEM((CH, DH), jnp.float32) for _ in range(NB)],
        pltpu.VMEM((ZROWS, DH), jnp.float32),
        [pltpu.SemaphoreType.DMA for _ in range(NB)],
        [pltpu.SemaphoreType.DMA for _ in range(NB)],
        [pltpu.SemaphoreType.DMA for _ in range(2)],
    ],
)
def _spmv_sc(ego2, colm2, rowm, valm, out, acc, colvs, rowvs, valvs,
             gbufs, zbuf, gsems, ssems, stsems):
    c = lax.axis_index("c")
    s = lax.axis_index("s")
    tile_crow = s * CROWS_PER_TILE

    def crow_of(m):
        return jnp.minimum(tile_crow + m * KR, MAX_CROW)

    def stage_descs(m, b):
        crow = crow_of(m)
        return (
            (colm2.at[c, pl.ds(crow, KR)], colvs[b], stsems[b]),
            (rowm.at[pl.ds(crow, KR)], rowvs[b], stsems[b]),
            (valm.at[pl.ds(crow * CH, KR * CH)], valvs[b], stsems[b]),
        )

    def stage_issue(m, b):
        for src, dst, sem in stage_descs(m, b):
            pltpu.async_copy(src, dst, sem)

    def stage_wait(m, b):
        for src, dst, sem in stage_descs(m, b):
            pltpu.make_async_copy(src, dst, sem).wait()

    def g0_issue(b):
        pltpu.async_copy(ego2.at[colvs[b].at[0]], gbufs[0], gsems[0])

    def g0_wait(b):
        pltpu.make_async_copy(ego2.at[colvs[b].at[0]], gbufs[0],
                              gsems[0]).wait()

    stage_issue(0, 0)
    stage_wait(0, 0)
    stage_issue(1, 1)
    g0_issue(0)

    zeros16 = jnp.zeros((16,), jnp.float32)

    @plsc.parallel_loop(0, ZROWS)
    def _(j):
        zbuf[j, pl.ds(0, 16)] = zeros16
        zbuf[j, pl.ds(16, 16)] = zeros16

    row0 = s * ROWS_PER_TILE

    def zero_body(i, _):
        pltpu.sync_copy(zbuf, acc.at[pl.ds(row0 + i * ZROWS, ZROWS)])
        return 0

    lax.fori_loop(0, ROWS_PER_TILE // ZROWS, zero_body, 0)

    plsc.subcore_barrier()

    def process(i, a, b):
        colv, rowv, valv = colvs[a], rowvs[a], valvs[a]
        gd = [None] * (KR + 1)
        sd = [None] * KR
        for k in range(KR):
            kb = k % NB
            if k + 1 < KR:
                nb = (k + 1) % NB
                if k + 1 >= NB:
                    sd[k + 1 - NB].wait()
                gd[k + 1] = pltpu.async_copy(ego2.at[colv.at[k + 1]],
                                             gbufs[nb], gsems[nb])
            if k == 0:
                g0_wait(a)
            else:
                gd[k].wait()
            g = gbufs[kb]

            @plsc.parallel_loop(0, CH // 16, unroll=2)
            def _(q):
                vv = valv[pl.ds(k * CH + q * 16, 16)]
                for j in range(16):
                    v = jnp.full((16,), vv[j])
                    e = q * 16 + j
                    g[e, pl.ds(0, 16)] = g[e, pl.ds(0, 16)] * v
                    g[e, pl.ds(16, 16)] = g[e, pl.ds(16, 16)] * v

            sd[k] = pltpu.async_copy(g, acc.at[rowv.at[k]], ssems[kb],
                                     add=True)
        for k in range(KR - NB, KR):
            sd[k].wait()
        stage_wait(i + 1, b)
        g0_issue(b)
        stage_issue(i + 2, a)

    def super_pair(j, _):
        process(2 * j, 0, 1)
        process(2 * j + 1, 1, 0)
        return 0

    lax.fori_loop(0, SUPER // 2, super_pair, 0)

    g0_wait(0)
    stage_wait(SUPER + 1, 1)

    plsc.subcore_barrier()

    pltpu.sync_copy(acc.at[pl.ds(row0, ROWS_PER_TILE)],
                    out.at[c, pl.ds(row0, ROWS_PER_TILE)])


BLK = 7168
GRID = N_PAD // BLK


def _lrelu(x):
    return jnp.where(x >= 0, x, 0.2 * x)


def _tc_compute(h0, h1, ego, wg, bg, wb, bb):
    side = jnp.concatenate([h0[0], h1[0]], axis=1)
    e = ego[...]
    o = (_lrelu(jnp.dot(side, wg[...], preferred_element_type=jnp.float32)
                + bg[...])
         + _lrelu(jnp.dot(e * side, wb[...], preferred_element_type=jnp.float32)
                  + bb[...]))
    ss = jnp.sum(o * o, axis=1, keepdims=True)
    nrm = o / (jnp.sqrt(ss) + 1e-12)
    return e, o, nrm


def _tc_body_first(h0, h1, ego, wg, bg, wb, bb, egon, out4):
    e, o, nrm = _tc_compute(h0, h1, ego, wg, bg, wb, bb)
    egon[...] = o
    out4[...] = jnp.stack([e, nrm], axis=0)


def _tc_body_rest(h0, h1, ego, wg, bg, wb, bb, prev4, egon, out4):
    del prev4
    e, o, nrm = _tc_compute(h0, h1, ego, wg, bg, wb, bb)
    egon[...] = o
    out4[...] = nrm[None]


_common_in_specs = [
    pl.BlockSpec((1, BLK, DH), lambda i: (0, i, 0)),
    pl.BlockSpec((1, BLK, DH), lambda i: (1, i, 0)),
    pl.BlockSpec((BLK, D), lambda i: (i, 0)),
    pl.BlockSpec((D, D), lambda i: (0, 0)),
    pl.BlockSpec((1, D), lambda i: (0, 0)),
    pl.BlockSpec((D, D), lambda i: (0, 0)),
    pl.BlockSpec((1, D), lambda i: (0, 0)),
]

_out_shapes = [
    jax.ShapeDtypeStruct((N_PAD, D), jnp.float32),
    jax.ShapeDtypeStruct((HOPS + 1, N, D), jnp.float32),
]


def _make_tc(kk):
    if kk == 0:
        return pl.pallas_call(
            _tc_body_first,
            grid=(GRID,),
            in_specs=_common_in_specs,
            out_specs=[
                pl.BlockSpec((BLK, D), lambda i: (i, 0)),
                pl.BlockSpec((2, BLK, D), lambda i: (0, i, 0)),
            ],
            out_shape=_out_shapes,
        )
    return pl.pallas_call(
        _tc_body_rest,
        grid=(GRID,),
        in_specs=_common_in_specs + [pl.BlockSpec(memory_space=pl.ANY)],
        out_specs=[
            pl.BlockSpec((BLK, D), lambda i: (i, 0)),
            pl.BlockSpec((1, BLK, D), lambda i, kk=kk: (kk + 1, i, 0)),
        ],
        out_shape=_out_shapes,
        input_output_aliases={7: 1},
    )


_tc_calls = [_make_tc(kk) for kk in range(HOPS)]


def kernel(user_emb, item_emb, adj_indices, adj_values,
           W_gc_0, b_gc_0, W_bi_0, b_bi_0,
           W_gc_1, b_gc_1, W_bi_1, b_bi_1,
           W_gc_2, b_gc_2, W_bi_2, b_bi_2):
    Ws_gc = [W_gc_0, W_gc_1, W_gc_2]
    bs_gc = [b_gc_0, b_gc_1, b_gc_2]
    Ws_bi = [W_bi_0, W_bi_1, W_bi_2]
    bs_bi = [b_bi_0, b_bi_1, b_bi_2]

    pad = E_PAD - E
    spread = jnp.arange(pad, dtype=jnp.int32) % N
    col_p = jnp.concatenate([adj_indices[1].astype(jnp.int32), spread])
    row_p = jnp.concatenate([adj_indices[0].astype(jnp.int32), spread])
    val_p = jnp.concatenate([adj_values, jnp.zeros((pad,), jnp.float32)])
    colm2 = (2 * col_p[None, :] +
             jnp.array([0, 1], jnp.int32)[:, None]).reshape(2, CROWS, CH)
    rowm = row_p.reshape(CROWS, CH)

    ego = jnp.concatenate([user_emb, item_emb], axis=0)
    egop = jnp.pad(ego, ((0, N_PAD - N), (0, 0)))
    out4 = None
    for k in range(HOPS):
        ego2 = egop.reshape(2 * N_PAD, DH)
        side2 = _spmv_sc(ego2, colm2, rowm, val_p)
        args = (side2, side2, egop,
                Ws_gc[k], bs_gc[k], Ws_bi[k], bs_bi[k])
        if k == 0:
            egop, out4 = _tc_calls[k](*args)
        else:
            egop, out4 = _tc_calls[k](*args, out4)
    return jnp.transpose(out4, (1, 0, 2))


# speedup vs baseline: 1.0029x; 1.0029x over previous
import functools

import jax
import jax.numpy as jnp
from jax import lax
from jax.experimental import pallas as pl
from jax.experimental.pallas import tpu as pltpu
from jax.experimental.pallas import tpu_sc as plsc

N = 50000
E = 800000
D = 64
DH = 32
HOPS = 3

NUM_CORES = 2
NUM_TILES = 16
CH = 128
KR = 16
SUPER = 26
NB = 4
CROWS_PER_TILE = KR * SUPER
CROWS = NUM_TILES * CROWS_PER_TILE
MAX_CROW = CROWS - KR
E_PAD = CROWS * CH
N_PAD = 50176
ROWS_PER_TILE = N_PAD // NUM_TILES
N_PAD4 = N_PAD // 4
ZROWS = 56

_mesh = plsc.VectorSubcoreMesh(core_axis_name="c", subcore_axis_name="s")


@functools.partial(
    pl.kernel,
    out_type=jax.ShapeDtypeStruct((NUM_CORES, N_PAD, DH), jnp.float32),
    mesh=_mesh,
    compiler_params=pltpu.CompilerParams(use_tc_tiling_on_sc=False),
    scratch_types=[
        pltpu.VMEM_SHARED((N_PAD, DH), jnp.float32),
        [pltpu.VMEM((KR, CH), jnp.int32) for _ in range(2)],
        [pltpu.VMEM((KR, CH), jnp.int32) for _ in range(2)],
        [pltpu.VMEM((KR * CH,), jnp.float32) for _ in range(2)],
        [pltpu.VMEM((CH, DH), jnp.float32) for _ in range(NB)],
        pltpu.VMEM((ZROWS, DH), jnp.float32),
        [pltpu.SemaphoreType.DMA for _ in range(NB)],
        [pltpu.SemaphoreType.DMA for _ in range(NB)],
        [pltpu.SemaphoreType.DMA for _ in range(2)],
    ],
)
def _spmv_sc(ego2, colm2, rowm, valm, out, acc, colvs, rowvs, valvs,
             gbufs, zbuf, gsems, ssems, stsems):
    c = lax.axis_index("c")
    s = lax.axis_index("s")
    tile_crow = s * CROWS_PER_TILE

    def crow_of(m):
        return jnp.minimum(tile_crow + m * KR, MAX_CROW)

    def stage_descs(m, b):
        crow = crow_of(m)
        return (
            (colm2.at[c, pl.ds(crow, KR)], colvs[b], stsems[b]),
            (rowm.at[pl.ds(crow, KR)], rowvs[b], stsems[b]),
            (valm.at[pl.ds(crow * CH, KR * CH)], valvs[b], stsems[b]),
        )

    def stage_issue(m, b):
        for src, dst, sem in stage_descs(m, b):
            pltpu.async_copy(src, dst, sem)

    def stage_wait(m, b):
        for src, dst, sem in stage_descs(m, b):
            pltpu.make_async_copy(src, dst, sem).wait()

    def g0_issue(b):
        pltpu.async_copy(ego2.at[colvs[b].at[0]], gbufs[0], gsems[0])

    def g0_wait(b):
        pltpu.make_async_copy(ego2.at[colvs[b].at[0]], gbufs[0],
                              gsems[0]).wait()

    stage_issue(0, 0)
    stage_wait(0, 0)
    stage_issue(1, 1)
    g0_issue(0)

    zeros16 = jnp.zeros((16,), jnp.float32)

    @plsc.parallel_loop(0, ZROWS)
    def _(j):
        zbuf[j, pl.ds(0, 16)] = zeros16
        zbuf[j, pl.ds(16, 16)] = zeros16

    row0 = s * ROWS_PER_TILE

    def zero_body(i, _):
        pltpu.sync_copy(zbuf, acc.at[pl.ds(row0 + i * ZROWS, ZROWS)])
        return 0

    lax.fori_loop(0, ROWS_PER_TILE // ZROWS, zero_body, 0)

    plsc.subcore_barrier()

    def process(i, a, b):
        colv, rowv, valv = colvs[a], rowvs[a], valvs[a]
        gd = [None] * (KR + 1)
        sd = [None] * KR
        for k in range(KR):
            kb = k % NB
            if k + 1 < KR:
                nb = (k + 1) % NB
                if k + 1 >= NB:
                    sd[k + 1 - NB].wait()
                gd[k + 1] = pltpu.async_copy(ego2.at[colv.at[k + 1]],
                                             gbufs[nb], gsems[nb])
            if k == 0:
                g0_wait(a)
            else:
                gd[k].wait()
            g = gbufs[kb]

            @plsc.parallel_loop(0, CH // 16, unroll=2)
            def _(q):
                vv = valv[pl.ds(k * CH + q * 16, 16)]
                for j in range(16):
                    v = jnp.full((16,), vv[j])
                    e = q * 16 + j
                    g[e, pl.ds(0, 16)] = g[e, pl.ds(0, 16)] * v
                    g[e, pl.ds(16, 16)] = g[e, pl.ds(16, 16)] * v

            sd[k] = pltpu.async_copy(g, acc.at[rowv.at[k]], ssems[kb],
                                     add=True)
        for k in range(KR - NB, KR):
            sd[k].wait()
        stage_wait(i + 1, b)
        g0_issue(b)
        stage_issue(i + 2, a)

    def super_pair(j, _):
        process(2 * j, 0, 1)
        process(2 * j + 1, 1, 0)
        return 0

    lax.fori_loop(0, SUPER // 2, super_pair, 0)

    g0_wait(0)
    stage_wait(SUPER + 1, 1)

    plsc.subcore_barrier()

    pltpu.sync_copy(acc.at[pl.ds(row0, ROWS_PER_TILE)],
                    out.at[c, pl.ds(row0, ROWS_PER_TILE)])


BLK = 3584
GRID = N_PAD // BLK


def _lrelu(x):
    return jnp.where(x >= 0, x, 0.2 * x)


def _tc_compute(h0, h1, ego, wg, bg, wb, bb):
    side = jnp.concatenate([h0[0], h1[0]], axis=1)
    e = ego[...]
    o = (_lrelu(jnp.dot(side, wg[...], preferred_element_type=jnp.float32)
                + bg[...])
         + _lrelu(jnp.dot(e * side, wb[...], preferred_element_type=jnp.float32)
                  + bb[...]))
    ss = jnp.sum(o * o, axis=1, keepdims=True)
    nrm = o / (jnp.sqrt(ss) + 1e-12)
    return e, o, nrm


def _tc_body_first(h0, h1, ego, wg, bg, wb, bb, egon, out4):
    e, o, nrm = _tc_compute(h0, h1, ego, wg, bg, wb, bb)
    egon[...] = o
    out4[...] = jnp.stack([e, nrm], axis=0)


def _tc_body_rest(h0, h1, ego, wg, bg, wb, bb, prev4, egon, out4):
    del prev4
    e, o, nrm = _tc_compute(h0, h1, ego, wg, bg, wb, bb)
    egon[...] = o
    out4[...] = nrm[None]


_common_in_specs = [
    pl.BlockSpec((1, BLK, DH), lambda i: (0, i, 0)),
    pl.BlockSpec((1, BLK, DH), lambda i: (1, i, 0)),
    pl.BlockSpec((BLK, D), lambda i: (i, 0)),
    pl.BlockSpec((D, D), lambda i: (0, 0)),
    pl.BlockSpec((1, D), lambda i: (0, 0)),
    pl.BlockSpec((D, D), lambda i: (0, 0)),
    pl.BlockSpec((1, D), lambda i: (0, 0)),
]

_out_shapes = [
    jax.ShapeDtypeStruct((N_PAD, D), jnp.float32),
    jax.ShapeDtypeStruct((HOPS + 1, N, D), jnp.float32),
]


def _make_tc(kk):
    if kk == 0:
        return pl.pallas_call(
            _tc_body_first,
            grid=(GRID,),
            in_specs=_common_in_specs,
            out_specs=[
                pl.BlockSpec((BLK, D), lambda i: (i, 0)),
                pl.BlockSpec((2, BLK, D), lambda i: (0, i, 0)),
            ],
            out_shape=_out_shapes,
        )
    return pl.pallas_call(
        _tc_body_rest,
        grid=(GRID,),
        in_specs=_common_in_specs + [pl.BlockSpec(memory_space=pl.ANY)],
        out_specs=[
            pl.BlockSpec((BLK, D), lambda i: (i, 0)),
            pl.BlockSpec((1, BLK, D), lambda i, kk=kk: (kk + 1, i, 0)),
        ],
        out_shape=_out_shapes,
        input_output_aliases={7: 1},
    )


_tc_calls = [_make_tc(kk) for kk in range(HOPS)]


def kernel(user_emb, item_emb, adj_indices, adj_values,
           W_gc_0, b_gc_0, W_bi_0, b_bi_0,
           W_gc_1, b_gc_1, W_bi_1, b_bi_1,
           W_gc_2, b_gc_2, W_bi_2, b_bi_2):
    Ws_gc = [W_gc_0, W_gc_1, W_gc_2]
    bs_gc = [b_gc_0, b_gc_1, b_gc_2]
    Ws_bi = [W_bi_0, W_bi_1, W_bi_2]
    bs_bi = [b_bi_0, b_bi_1, b_bi_2]

    pad = E_PAD - E
    spread = jnp.arange(pad, dtype=jnp.int32) % N
    col_p = jnp.concatenate([adj_indices[1].astype(jnp.int32), spread])
    row_p = jnp.concatenate([adj_indices[0].astype(jnp.int32), spread])
    val_p = jnp.concatenate([adj_values, jnp.zeros((pad,), jnp.float32)])
    colm2 = (2 * col_p[None, :] +
             jnp.array([0, 1], jnp.int32)[:, None]).reshape(2, CROWS, CH)
    rowm = row_p.reshape(CROWS, CH)

    ego = jnp.concatenate([user_emb, item_emb], axis=0)
    egop = jnp.pad(ego, ((0, N_PAD - N), (0, 0)))
    out4 = None
    for k in range(HOPS):
        ego2 = egop.reshape(2 * N_PAD, DH)
        side2 = _spmv_sc(ego2, colm2, rowm, val_p)
        args = (side2, side2, egop,
                Ws_gc[k], bs_gc[k], Ws_bi[k], bs_bi[k])
        if k == 0:
            egop, out4 = _tc_calls[k](*args)
        else:
            egop, out4 = _tc_calls[k](*args, out4)
    return jnp.transpose(out4, (1, 0, 2))
